# baseline (speedup 1.0000x reference)
import functools

import jax
import jax.numpy as jnp
from jax import lax
from jax.experimental import pallas as pl
from jax.experimental.pallas import tpu as pltpu
from jax.experimental.pallas import tpu_sc as plsc

S = 200
B = 4096
D = 32
NTOK = S * B
CHUNK = 128


def _gather_sc(emb, x2):
    info = plsc.get_sparse_core_info()
    nw = info.num_cores * info.num_subcores
    jrows = (NTOK // CHUNK) // nw
    mesh = plsc.VectorSubcoreMesh(core_axis_name="c", subcore_axis_name="s")

    @functools.partial(
        pl.kernel,
        mesh=mesh,
        compiler_params=pltpu.CompilerParams(use_tc_tiling_on_sc=False),
        out_type=jax.ShapeDtypeStruct((NTOK, D), jnp.float32),
        scratch_types=[
            pltpu.VMEM((jrows, CHUNK), jnp.int32),
            pltpu.VMEM((CHUNK, D), jnp.float32),
            pltpu.SemaphoreType.DMA,
        ],
    )
    def k(table_hbm, idx_hbm, out_hbm, idx_v, rows_v, sem):
        wid = lax.axis_index("s") * info.num_cores + lax.axis_index("c")
        base_j = wid * jrows
        pltpu.sync_copy(idx_hbm.at[pl.ds(base_j, jrows)], idx_v)

        def body(j, carry):
            pltpu.async_copy(table_hbm.at[idx_v.at[j]], rows_v, sem).wait()
            pltpu.sync_copy(rows_v, out_hbm.at[pl.ds((base_j + j) * CHUNK, CHUNK)])
            return carry

        lax.fori_loop(0, jrows, body, 0)

    return k(emb, x2)


def _head_tc(e3, wt, b2):
    def body(e_ref, w_ref, b_ref, o_ref):
        e = e_ref[0]
        m = jnp.max(e, axis=0, keepdims=True)
        ex = jnp.exp(e - m)
        exm = jnp.where(e != 0.0, ex, 0.0)
        ssum = jnp.sum(exm, axis=0, keepdims=True)
        a = exm / ssum
        pooled = jnp.sum(a, axis=0, keepdims=True) * (1.0 / B)
        o_ref[0] = jnp.dot(pooled, w_ref[...],
                           preferred_element_type=jnp.float32) + b_ref[...]

    return pl.pallas_call(
        body,
        grid=(S,),
        in_specs=[
            pl.BlockSpec((1, B, D), lambda i: (i, 0, 0)),
            pl.BlockSpec((D, 2), lambda i: (0, 0)),
            pl.BlockSpec((1, 2), lambda i: (0, 0)),
        ],
        out_specs=pl.BlockSpec((1, 1, 2), lambda i: (i, 0, 0)),
        out_shape=jax.ShapeDtypeStruct((S, 1, 2), jnp.float32),
    )(e3, wt, b2)


def kernel(x, emb, W, b):
    x2 = x.astype(jnp.int32).reshape(NTOK // CHUNK, CHUNK)
    e_flat = _gather_sc(emb, x2)
    out = _head_tc(e_flat.reshape(S, B, D), W.T, b.reshape(1, 2))
    return out.reshape(S, 2)


# speedup vs baseline: 1.5691x; 1.5691x over previous
import functools

import jax
import jax.numpy as jnp
from jax import lax
from jax.experimental import pallas as pl
from jax.experimental.pallas import tpu as pltpu
from jax.experimental.pallas import tpu_sc as plsc

S = 200
B = 4096
D = 32
L = 16
CHUNK = 128
NCH = B // CHUNK
NBUF = 2
UNROLL = 4
NEG = -1e30


def _fused_sc(emb, x3):
    info = plsc.get_sparse_core_info()
    nw = info.num_cores * info.num_subcores
    base_seg, extra = S // nw, S % nw
    mesh = plsc.VectorSubcoreMesh(core_axis_name="c", subcore_axis_name="s")

    @functools.partial(
        pl.kernel,
        mesh=mesh,
        compiler_params=pltpu.CompilerParams(use_tc_tiling_on_sc=False),
        out_type=jax.ShapeDtypeStruct((S, D), jnp.float32),
        scratch_types=[
            pltpu.VMEM((NCH, CHUNK), jnp.int32),
            pltpu.VMEM((NBUF, CHUNK, D), jnp.float32),
            pltpu.VMEM((B, L), jnp.uint32),
            pltpu.VMEM((D,), jnp.float32),
            pltpu.SemaphoreType.DMA,
            pltpu.SemaphoreType.DMA,
        ],
    )
    def k(table, idx_hbm, out_hbm, idx_v, ring, ebf, rowbuf, sem0, sem1):
        sems = (sem0, sem1)
        wid = lax.axis_index("s") * info.num_cores + lax.axis_index("c")
        nseg = jnp.where(wid < extra, base_seg + 1, base_seg)
        seg0 = base_seg * wid + jnp.minimum(wid, extra)

        def fire(c, p):
            pltpu.async_copy(table.at[idx_v.at[c]], ring.at[p], sems[p])

        def drain(p):
            pltpu.make_async_copy(
                table.at[pl.ds(0, CHUNK)], ring.at[p], sems[p]
            ).wait()

        neg_inf = jnp.full((L,), -3.4e38, jnp.float32)
        zeros = jnp.zeros((L,), jnp.float32)
        himask = jnp.full((L,), 0xFFFF0000, jnp.uint32)

        def bf16_pack(lo, hi):
            lo_u = lax.bitcast_convert_type(lo, jnp.uint32)
            hi_u = lax.bitcast_convert_type(hi, jnp.uint32)
            return (lo_u >> 16) | (hi_u & himask)

        def bf16_unpack(u):
            lo = lax.bitcast_convert_type(u << 16, jnp.float32)
            hi = lax.bitcast_convert_type(u & himask, jnp.float32)
            return lo, hi

        def do_segment(s):
            pltpu.sync_copy(idx_hbm.at[s], idx_v)
            for p in range(NBUF):
                fire(p, p)

            def pair(j, m):
                for p in range(NBUF):
                    c = NBUF * j + p
                    drain(p)
                    rbuf = ring.at[p]

                    def row_a(r4, mm, _c=c, _rbuf=rbuf):
                        ml, mh = mm
                        for u in range(UNROLL):
                            r = UNROLL * r4 + u
                            lo = _rbuf[r, pl.ds(0, L)]
                            hi = _rbuf[r, pl.ds(L, L)]
                            ml = jnp.maximum(ml, lo)
                            mh = jnp.maximum(mh, hi)
                            mlo = jnp.where(lo == 0.0, NEG, lo)
                            mhi = jnp.where(hi == 0.0, NEG, hi)
                            ebf[_c * CHUNK + r] = bf16_pack(mlo, mhi)
                        return ml, mh

                    m = lax.fori_loop(0, CHUNK // UNROLL, row_a, m)

                    @pl.when(c + NBUF < NCH)
                    def _(_c=c, _p=p):
                        fire(_c + NBUF, _p)

                return m

            m_lo, m_hi = lax.fori_loop(0, NCH // NBUF, pair, (neg_inf, neg_inf))

            def row_b(r4, ss):
                sl, sh = ss
                for u in range(UNROLL):
                    r = UNROLL * r4 + u
                    a, bb = bf16_unpack(ebf[r])
                    sl = sl + jnp.exp(a - m_lo)
                    sh = sh + jnp.exp(bb - m_hi)
                return sl, sh

            s_lo, s_hi = lax.fori_loop(0, B // UNROLL, row_b, (zeros, zeros))
            inv_lo = 1.0 / s_lo
            inv_hi = 1.0 / s_hi

            def row_c(r4, pp):
                pacc_lo, pacc_hi = pp
                for u in range(UNROLL):
                    r = UNROLL * r4 + u
                    a, bb = bf16_unpack(ebf[r])
                    pacc_lo = pacc_lo + jnp.exp(a - m_lo) * inv_lo
                    pacc_hi = pacc_hi + jnp.exp(bb - m_hi) * inv_hi
                return pacc_lo, pacc_hi

            p_lo, p_hi = lax.fori_loop(0, B // UNROLL, row_c, (zeros, zeros))
            rowbuf[pl.ds(0, L)] = p_lo * (1.0 / B)
            rowbuf[pl.ds(L, L)] = p_hi * (1.0 / B)
            pltpu.sync_copy(rowbuf, out_hbm.at[s])

        for ki in range(base_seg + 1):
            @pl.when(ki < nseg)
            def _(_ki=ki):
                do_segment(seg0 + _ki)

    return k(emb, x3)


def _linear_tc(pooled, wt, b2):
    def body(p_ref, w_ref, b_ref, o_ref):
        o_ref[...] = (
            jnp.dot(p_ref[...], w_ref[...], preferred_element_type=jnp.float32)
            + b_ref[...]
        )

    return pl.pallas_call(
        body,
        out_shape=jax.ShapeDtypeStruct((S, 2), jnp.float32),
    )(pooled, wt, b2)


def kernel(x, emb, W, b):
    x3 = x.astype(jnp.int32).reshape(S, NCH, CHUNK)
    pooled = _fused_sc(emb, x3)
    return _linear_tc(pooled, W.T, b.reshape(1, 2))
